# Initial kernel scaffold; baseline (speedup 1.0000x reference)
import jax
import jax.numpy as jnp
from jax.experimental import pallas as pl


def kernel(x, off_w1, off_b1, off_w2, off_b2, shp_w1, shp_b1, shp_w2, shp_b2, cls_w1, cls_b1, cls_w2, cls_b2):
    raise NotImplementedError("write your pallas kernel here")


# speedup vs baseline: 1.3388x; 1.3388x over previous
import jax
import jax.numpy as jnp
from jax.experimental import pallas as pl
from jax.experimental.pallas import tpu as pltpu

B, C, H, W = 8, 96, 128, 128
NUM_CLASSES = 80
HID = 3 * C
OUT_CH = NUM_CLASSES + 4
HC = 16


def _fused_head_kernel(xt_ref, w1_ref, b1_ref, w2_ref, b2_ref,
                       cls_ref, off_ref, shp_ref, xp_ref, acc_ref):
    xp_ref[0, :, :] = jnp.zeros((W + 2, C), jnp.bfloat16)
    xp_ref[H + 1, :, :] = jnp.zeros((W + 2, C), jnp.bfloat16)
    xp_ref[:, 0, :] = jnp.zeros((H + 2, C), jnp.bfloat16)
    xp_ref[:, W + 1, :] = jnp.zeros((H + 2, C), jnp.bfloat16)
    xp_ref[1:H + 1, 1:W + 1, :] = xt_ref[0]

    b1 = b1_ref[0][None, :]
    b2 = b2_ref[0][None, :]
    for h0 in range(0, H, HC):
        for k in range(9):
            dy, dx = k // 3, k % 3
            slab = xp_ref[h0 + dy:h0 + dy + HC, dx:dx + W, :]
            mm = jax.lax.dot_general(
                slab.reshape(HC * W, C), w1_ref[k],
                (((1,), (0,)), ((), ())),
                preferred_element_type=jnp.float32)
            if k == 0:
                acc_ref[:, :] = mm
            else:
                acc_ref[:, :] = acc_ref[:, :] + mm
        hid = jnp.maximum(acc_ref[:, :] + b1, 0.0).astype(jnp.bfloat16)
        out = jax.lax.dot_general(
            hid, w2_ref[:, :], (((1,), (0,)), ((), ())),
            preferred_element_type=jnp.float32) + b2
        out3 = out.reshape(HC, W, OUT_CH)
        cls_ref[0, h0:h0 + HC] = out3[:, :, :NUM_CLASSES]
        off_ref[0, h0:h0 + HC] = out3[:, :, NUM_CLASSES:NUM_CLASSES + 2].reshape(HC, 2 * W)
        shp_ref[0, h0:h0 + HC] = out3[:, :, NUM_CLASSES + 2:NUM_CLASSES + 4].reshape(HC, 2 * W)


def kernel(x, off_w1, off_b1, off_w2, off_b2, shp_w1, shp_b1, shp_w2, shp_b2,
           cls_w1, cls_b1, cls_w2, cls_b2):
    xt = jnp.transpose(x, (0, 2, 3, 1)).astype(jnp.bfloat16)
    w1_cat = jnp.concatenate([cls_w1, off_w1, shp_w1], axis=0)
    w1_r = jnp.transpose(w1_cat, (2, 3, 1, 0)).reshape(9, C, HID).astype(jnp.bfloat16)
    b1_cat = jnp.concatenate([cls_b1, off_b1, shp_b1])[None, :]
    w2 = jnp.zeros((HID, OUT_CH), jnp.float32)
    w2 = w2.at[0:C, 0:NUM_CLASSES].set(cls_w2[:, :, 0, 0].T)
    w2 = w2.at[C:2 * C, NUM_CLASSES:NUM_CLASSES + 2].set(off_w2[:, :, 0, 0].T)
    w2 = w2.at[2 * C:3 * C, NUM_CLASSES + 2:].set(shp_w2[:, :, 0, 0].T)
    w2 = w2.astype(jnp.bfloat16)
    b2_cat = jnp.concatenate([cls_b2, off_b2, shp_b2])[None, :]

    cls, off, shp = pl.pallas_call(
        _fused_head_kernel,
        grid=(B,),
        in_specs=[
            pl.BlockSpec((1, H, W, C), lambda b: (b, 0, 0, 0)),
            pl.BlockSpec((9, C, HID), lambda b: (0, 0, 0)),
            pl.BlockSpec((1, HID), lambda b: (0, 0)),
            pl.BlockSpec((HID, OUT_CH), lambda b: (0, 0)),
            pl.BlockSpec((1, OUT_CH), lambda b: (0, 0)),
        ],
        out_specs=[
            pl.BlockSpec((1, H, W, NUM_CLASSES), lambda b: (b, 0, 0, 0)),
            pl.BlockSpec((1, H, 2 * W), lambda b: (b, 0, 0)),
            pl.BlockSpec((1, H, 2 * W), lambda b: (b, 0, 0)),
        ],
        out_shape=[
            jax.ShapeDtypeStruct((B, H, W, NUM_CLASSES), jnp.float32),
            jax.ShapeDtypeStruct((B, H, 2 * W), jnp.float32),
            jax.ShapeDtypeStruct((B, H, 2 * W), jnp.float32),
        ],
        scratch_shapes=[
            pltpu.VMEM((H + 2, W + 2, C), jnp.bfloat16),
            pltpu.VMEM((HC * W, HID), jnp.float32),
        ],
    )(xt, w1_r, b1_cat, w2, b2_cat)
    return cls, off.reshape(B, H, W, 2), shp.reshape(B, H, W, 2)


# speedup vs baseline: 1.7265x; 1.2896x over previous
import jax
import jax.numpy as jnp
from jax.experimental import pallas as pl
from jax.experimental.pallas import tpu as pltpu

B, C, H, W = 8, 96, 128, 128
NUM_CLASSES = 80
CP = 128
KK = 3 * CP
HID = 3 * C
OUT_CH = NUM_CLASSES + 4
HC = 32


def _fused_head_kernel(xt_ref, w1_ref, b1_ref, w2_ref, b2_ref,
                       cls_ref, off_ref, shp_ref, xc_ref, acc_ref):
    zrow = jnp.zeros((W, KK), jnp.bfloat16)
    xc_ref[0] = zrow
    xc_ref[H + 1] = zrow
    zstrip = jnp.zeros((H, W, CP - C), jnp.bfloat16)
    for dx in range(3):
        xc_ref[1:H + 1, :, dx * CP + C:(dx + 1) * CP] = zstrip
    x = xt_ref[0]
    xc_ref[1:H + 1, 1:W, 0:C] = x[:, 0:W - 1, :]
    xc_ref[1:H + 1, 0:1, 0:C] = jnp.zeros((H, 1, C), jnp.bfloat16)
    xc_ref[1:H + 1, :, CP:CP + C] = x
    xc_ref[1:H + 1, 0:W - 1, 2 * CP:2 * CP + C] = x[:, 1:W, :]
    xc_ref[1:H + 1, W - 1:W, 2 * CP:2 * CP + C] = jnp.zeros((H, 1, C), jnp.bfloat16)

    b1 = b1_ref[0][None, :]
    b2 = b2_ref[0][None, :]
    for h0 in range(0, H, HC):
        for dy in range(3):
            slab = xc_ref[h0 + dy:h0 + dy + HC].reshape(HC * W, KK)
            mm = jax.lax.dot_general(
                slab, w1_ref[dy],
                (((1,), (0,)), ((), ())),
                preferred_element_type=jnp.float32)
            if dy == 0:
                acc_ref[:, :] = mm
            else:
                acc_ref[:, :] = acc_ref[:, :] + mm
        hid = jnp.maximum(acc_ref[:, :] + b1, 0.0).astype(jnp.bfloat16)
        out = jax.lax.dot_general(
            hid, w2_ref[:, :], (((1,), (0,)), ((), ())),
            preferred_element_type=jnp.float32) + b2
        out3 = out.reshape(HC, W, OUT_CH)
        cls_ref[0, h0:h0 + HC] = out3[:, :, :NUM_CLASSES]
        off_ref[0, h0:h0 + HC] = out3[:, :, NUM_CLASSES:NUM_CLASSES + 2].reshape(HC, 2 * W)
        shp_ref[0, h0:h0 + HC] = out3[:, :, NUM_CLASSES + 2:NUM_CLASSES + 4].reshape(HC, 2 * W)


def kernel(x, off_w1, off_b1, off_w2, off_b2, shp_w1, shp_b1, shp_w2, shp_b2,
           cls_w1, cls_b1, cls_w2, cls_b2):
    xt = jnp.transpose(x, (0, 2, 3, 1)).astype(jnp.bfloat16)
    w1_cat = jnp.concatenate([cls_w1, off_w1, shp_w1], axis=0)
    w1_k = jnp.transpose(w1_cat, (2, 3, 1, 0))
    w1_r = jnp.zeros((3, KK, HID), jnp.float32)
    for dx in range(3):
        w1_r = w1_r.at[:, dx * CP:dx * CP + C, :].set(w1_k[:, dx])
    w1_r = w1_r.astype(jnp.bfloat16)
    b1_cat = jnp.concatenate([cls_b1, off_b1, shp_b1])[None, :]
    w2 = jnp.zeros((HID, OUT_CH), jnp.float32)
    w2 = w2.at[0:C, 0:NUM_CLASSES].set(cls_w2[:, :, 0, 0].T)
    w2 = w2.at[C:2 * C, NUM_CLASSES:NUM_CLASSES + 2].set(off_w2[:, :, 0, 0].T)
    w2 = w2.at[2 * C:3 * C, NUM_CLASSES + 2:].set(shp_w2[:, :, 0, 0].T)
    w2 = w2.astype(jnp.bfloat16)
    b2_cat = jnp.concatenate([cls_b2, off_b2, shp_b2])[None, :]

    cls, off, shp = pl.pallas_call(
        _fused_head_kernel,
        grid=(B,),
        in_specs=[
            pl.BlockSpec((1, H, W, C), lambda b: (b, 0, 0, 0)),
            pl.BlockSpec((3, KK, HID), lambda b: (0, 0, 0)),
            pl.BlockSpec((1, HID), lambda b: (0, 0)),
            pl.BlockSpec((HID, OUT_CH), lambda b: (0, 0)),
            pl.BlockSpec((1, OUT_CH), lambda b: (0, 0)),
        ],
        out_specs=[
            pl.BlockSpec((1, H, W, NUM_CLASSES), lambda b: (b, 0, 0, 0)),
            pl.BlockSpec((1, H, 2 * W), lambda b: (b, 0, 0)),
            pl.BlockSpec((1, H, 2 * W), lambda b: (b, 0, 0)),
        ],
        out_shape=[
            jax.ShapeDtypeStruct((B, H, W, NUM_CLASSES), jnp.float32),
            jax.ShapeDtypeStruct((B, H, 2 * W), jnp.float32),
            jax.ShapeDtypeStruct((B, H, 2 * W), jnp.float32),
        ],
        scratch_shapes=[
            pltpu.VMEM((H + 2, W, KK), jnp.bfloat16),
            pltpu.VMEM((HC * W, HID), jnp.float32),
        ],
    )(xt, w1_r, b1_cat, w2, b2_cat)
    return cls, off.reshape(B, H, W, 2), shp.reshape(B, H, W, 2)


# speedup vs baseline: 2.5025x; 1.4494x over previous
import jax
import jax.numpy as jnp
from jax.experimental import pallas as pl
from jax.experimental.pallas import tpu as pltpu

B, C, H, W = 8, 96, 128, 128
HW = H * W
NUM_CLASSES = 80
CP = 128
KK = 3 * CP
HID = 3 * C
OUT_CH = NUM_CLASSES + 4
MCH = 8192


def _fused_head_kernel(xb_ref, m_ref, w1_ref, b1_ref, w2_ref, b2_ref,
                       cls_ref, off_ref, shp_ref, xc_ref, acc_ref):
    @pl.when(pl.program_id(0) == 0)
    def _init():
        xc_ref[:, 0:W] = jnp.zeros((KK, W), jnp.bfloat16)
        xc_ref[:, W + HW:2 * W + HW] = jnp.zeros((KK, W), jnp.bfloat16)
        zstrip = jnp.zeros((CP - C, 2 * W + HW), jnp.bfloat16)
        for dx in range(3):
            xc_ref[dx * CP + C:(dx + 1) * CP, :] = zstrip
        xc_ref[0:C, W:W + 1] = jnp.zeros((C, 1), jnp.bfloat16)
        xc_ref[2 * CP:2 * CP + C, W + HW - 1:W + HW] = jnp.zeros((C, 1), jnp.bfloat16)

    xin = xb_ref[0]
    xc_ref[CP:CP + C, W:W + HW] = xin
    xc_ref[0:C, W + 1:W + HW] = xin[:, 0:HW - 1] * m_ref[0:1, 1:HW]
    xc_ref[2 * CP:2 * CP + C, W:W + HW - 1] = xin[:, 1:HW] * m_ref[1:2, 0:HW - 1]

    b1 = b1_ref[:, 0:1]
    b2 = b2_ref[:, 0:1]
    for m0 in range(0, HW, MCH):
        for dy in range(3):
            slab = xc_ref[:, dy * W + m0:dy * W + m0 + MCH]
            mm = jax.lax.dot_general(
                w1_ref[dy], slab, (((1,), (0,)), ((), ())),
                preferred_element_type=jnp.float32)
            if dy == 0:
                acc_ref[:, :] = mm
            else:
                acc_ref[:, :] = acc_ref[:, :] + mm
        hid = jnp.maximum(acc_ref[:, :] + b1, 0.0).astype(jnp.bfloat16)
        out = jax.lax.dot_general(
            w2_ref[:, :], hid, (((1,), (0,)), ((), ())),
            preferred_element_type=jnp.float32) + b2
        cls_ref[0, :, m0:m0 + MCH] = out[0:NUM_CLASSES]
        off_ref[0, :, m0:m0 + MCH] = out[NUM_CLASSES:NUM_CLASSES + 2]
        shp_ref[0, :, m0:m0 + MCH] = out[NUM_CLASSES + 2:NUM_CLASSES + 4]


def kernel(x, off_w1, off_b1, off_w2, off_b2, shp_w1, shp_b1, shp_w2, shp_b2,
           cls_w1, cls_b1, cls_w2, cls_b2):
    xb = x.reshape(B, C, HW).astype(jnp.bfloat16)
    p = jnp.arange(HW, dtype=jnp.int32)
    m = jnp.stack([(p % W != 0), (p % W != W - 1)]).astype(jnp.bfloat16)
    w1_cat = jnp.concatenate([cls_w1, off_w1, shp_w1], axis=0)
    w1_k = jnp.transpose(w1_cat, (2, 3, 1, 0))
    w1_r = jnp.zeros((3, KK, HID), jnp.float32)
    for dx in range(3):
        w1_r = w1_r.at[:, dx * CP:dx * CP + C, :].set(w1_k[:, dx])
    w1_r = jnp.transpose(w1_r, (0, 2, 1)).astype(jnp.bfloat16)
    b1_col = jnp.concatenate([cls_b1, off_b1, shp_b1])[:, None]
    w2 = jnp.zeros((OUT_CH, HID), jnp.float32)
    w2 = w2.at[0:NUM_CLASSES, 0:C].set(cls_w2[:, :, 0, 0])
    w2 = w2.at[NUM_CLASSES:NUM_CLASSES + 2, C:2 * C].set(off_w2[:, :, 0, 0])
    w2 = w2.at[NUM_CLASSES + 2:, 2 * C:3 * C].set(shp_w2[:, :, 0, 0])
    w2 = w2.astype(jnp.bfloat16)
    b2_col = jnp.concatenate([cls_b2, off_b2, shp_b2])[:, None]

    cls, off, shp = pl.pallas_call(
        _fused_head_kernel,
        grid=(B,),
        in_specs=[
            pl.BlockSpec((1, C, HW), lambda b: (b, 0, 0)),
            pl.BlockSpec((2, HW), lambda b: (0, 0)),
            pl.BlockSpec((3, HID, KK), lambda b: (0, 0, 0)),
            pl.BlockSpec((HID, 1), lambda b: (0, 0)),
            pl.BlockSpec((OUT_CH, HID), lambda b: (0, 0)),
            pl.BlockSpec((OUT_CH, 1), lambda b: (0, 0)),
        ],
        out_specs=[
            pl.BlockSpec((1, NUM_CLASSES, HW), lambda b: (b, 0, 0)),
            pl.BlockSpec((1, 2, HW), lambda b: (b, 0, 0)),
            pl.BlockSpec((1, 2, HW), lambda b: (b, 0, 0)),
        ],
        out_shape=[
            jax.ShapeDtypeStruct((B, NUM_CLASSES, HW), jnp.float32),
            jax.ShapeDtypeStruct((B, 2, HW), jnp.float32),
            jax.ShapeDtypeStruct((B, 2, HW), jnp.float32),
        ],
        scratch_shapes=[
            pltpu.VMEM((KK, 2 * W + HW), jnp.bfloat16),
            pltpu.VMEM((HID, MCH), jnp.float32),
        ],
    )(xb, m, w1_r, b1_col, w2, b2_col)
    cls = jnp.transpose(cls.reshape(B, NUM_CLASSES, H, W), (0, 2, 3, 1))
    off = jnp.transpose(off.reshape(B, 2, H, W), (0, 2, 3, 1))
    shp = jnp.transpose(shp.reshape(B, 2, H, W), (0, 2, 3, 1))
    return cls, off, shp


# speedup vs baseline: 2.6812x; 1.0714x over previous
import jax
import jax.numpy as jnp
from jax.experimental import pallas as pl
from jax.experimental.pallas import tpu as pltpu

B, C, H, W = 8, 96, 128, 128
HW = H * W
NUM_CLASSES = 80
CP = 128
KK = 3 * CP
HID = 3 * C
OUT_CH = NUM_CLASSES + 4
MCH = 8192


def _fused_head_kernel(xb_ref, m_ref, w1_ref, b1_ref, w2c_ref, w2o_ref,
                       w2s_ref, b2_ref, cls_ref, off_ref, shp_ref,
                       xc_ref, acc_ref):
    @pl.when(pl.program_id(0) == 0)
    def _init():
        xc_ref[:, 0:W] = jnp.zeros((KK, W), jnp.bfloat16)
        xc_ref[:, W + HW:2 * W + HW] = jnp.zeros((KK, W), jnp.bfloat16)
        zstrip = jnp.zeros((CP - C, 2 * W + HW), jnp.bfloat16)
        for dx in range(3):
            xc_ref[dx * CP + C:(dx + 1) * CP, :] = zstrip
        xc_ref[0:C, W:W + 1] = jnp.zeros((C, 1), jnp.bfloat16)
        xc_ref[2 * CP:2 * CP + C, W + HW - 1:W + HW] = jnp.zeros((C, 1), jnp.bfloat16)

    xin = xb_ref[0].astype(jnp.bfloat16)
    xc_ref[CP:CP + C, W:W + HW] = xin
    xc_ref[0:C, W + 1:W + HW] = xin[:, 0:HW - 1] * m_ref[0:1, 1:HW]
    xc_ref[2 * CP:2 * CP + C, W:W + HW - 1] = xin[:, 1:HW] * m_ref[1:2, 0:HW - 1]

    b1 = b1_ref[:, 0:1]
    for m0 in range(0, HW, MCH):
        for dy in range(3):
            slab = xc_ref[:, dy * W + m0:dy * W + m0 + MCH]
            mm = jax.lax.dot_general(
                w1_ref[dy], slab, (((1,), (0,)), ((), ())),
                preferred_element_type=jnp.float32)
            if dy == 0:
                acc_ref[:, :] = mm
            else:
                acc_ref[:, :] = acc_ref[:, :] + mm
        hid = jnp.maximum(acc_ref[:, :] + b1, 0.0).astype(jnp.bfloat16)
        dn = (((1,), (0,)), ((), ()))
        cls_ref[0, :, m0:m0 + MCH] = jax.lax.dot_general(
            w2c_ref[:, :], hid[0:C], dn,
            preferred_element_type=jnp.float32) + b2_ref[0:NUM_CLASSES, 0:1]
        off_ref[0, :, m0:m0 + MCH] = jax.lax.dot_general(
            w2o_ref[:, :], hid[C:2 * C], dn,
            preferred_element_type=jnp.float32) + b2_ref[NUM_CLASSES:NUM_CLASSES + 2, 0:1]
        shp_ref[0, :, m0:m0 + MCH] = jax.lax.dot_general(
            w2s_ref[:, :], hid[2 * C:3 * C], dn,
            preferred_element_type=jnp.float32) + b2_ref[NUM_CLASSES + 2:, 0:1]


def kernel(x, off_w1, off_b1, off_w2, off_b2, shp_w1, shp_b1, shp_w2, shp_b2,
           cls_w1, cls_b1, cls_w2, cls_b2):
    xb = x.reshape(B, C, HW)
    p = jnp.arange(HW, dtype=jnp.int32)
    m = jnp.stack([(p % W != 0), (p % W != W - 1)]).astype(jnp.bfloat16)
    w1_cat = jnp.concatenate([cls_w1, off_w1, shp_w1], axis=0)
    w1_k = jnp.transpose(w1_cat, (2, 3, 1, 0))
    w1_r = jnp.zeros((3, KK, HID), jnp.float32)
    for dx in range(3):
        w1_r = w1_r.at[:, dx * CP:dx * CP + C, :].set(w1_k[:, dx])
    w1_r = jnp.transpose(w1_r, (0, 2, 1)).astype(jnp.bfloat16)
    b1_col = jnp.concatenate([cls_b1, off_b1, shp_b1])[:, None]
    w2c = cls_w2[:, :, 0, 0].astype(jnp.bfloat16)
    w2o = off_w2[:, :, 0, 0].astype(jnp.bfloat16)
    w2s = shp_w2[:, :, 0, 0].astype(jnp.bfloat16)
    b2_col = jnp.concatenate([cls_b2, off_b2, shp_b2])[:, None]

    cls, off, shp = pl.pallas_call(
        _fused_head_kernel,
        grid=(B,),
        in_specs=[
            pl.BlockSpec((1, C, HW), lambda b: (b, 0, 0)),
            pl.BlockSpec((2, HW), lambda b: (0, 0)),
            pl.BlockSpec((3, HID, KK), lambda b: (0, 0, 0)),
            pl.BlockSpec((HID, 1), lambda b: (0, 0)),
            pl.BlockSpec((NUM_CLASSES, C), lambda b: (0, 0)),
            pl.BlockSpec((2, C), lambda b: (0, 0)),
            pl.BlockSpec((2, C), lambda b: (0, 0)),
            pl.BlockSpec((OUT_CH, 1), lambda b: (0, 0)),
        ],
        out_specs=[
            pl.BlockSpec((1, NUM_CLASSES, HW), lambda b: (b, 0, 0)),
            pl.BlockSpec((1, 2, HW), lambda b: (b, 0, 0)),
            pl.BlockSpec((1, 2, HW), lambda b: (b, 0, 0)),
        ],
        out_shape=[
            jax.ShapeDtypeStruct((B, NUM_CLASSES, HW), jnp.float32),
            jax.ShapeDtypeStruct((B, 2, HW), jnp.float32),
            jax.ShapeDtypeStruct((B, 2, HW), jnp.float32),
        ],
        scratch_shapes=[
            pltpu.VMEM((KK, 2 * W + HW), jnp.bfloat16),
            pltpu.VMEM((HID, MCH), jnp.float32),
        ],
    )(xb, m, w1_r, b1_col, w2c, w2o, w2s, b2_col)
    cls = jnp.transpose(cls.reshape(B, NUM_CLASSES, H, W), (0, 2, 3, 1))
    off = jnp.transpose(off.reshape(B, 2, H, W), (0, 2, 3, 1))
    shp = jnp.transpose(shp.reshape(B, 2, H, W), (0, 2, 3, 1))
    return cls, off, shp
